# initial kernel scaffold (unmeasured)
import jax
import jax.numpy as jnp
from jax import lax
from jax.experimental import pallas as pl
from jax.experimental.pallas import tpu as pltpu


def kernel(
    x,
):
    def body(*refs):
        pass

    out_shape = jax.ShapeDtypeStruct(..., jnp.float32)
    return pl.pallas_call(body, out_shape=out_shape)(...)


# baseline (device time: 225731 ns/iter reference)
import jax
import jax.numpy as jnp
from jax import lax
from jax.experimental import pallas as pl
from jax.experimental.pallas import tpu as pltpu

M_PER = 8192
N_FULL = 2048
N_HALF = 1024
Q = 4096
R = 2048


def kernel(x):
    def body(x_hbm, out_ref, stage, send_buf, copy_sem, sx, rx, sy, ry):
        px = lax.axis_index("x")
        py = lax.axis_index("y")
        ox = 1 - px
        oy = 1 - py

        barrier = pltpu.get_barrier_semaphore()
        pl.semaphore_signal(barrier, inc=1, device_id=(ox, py),
                            device_id_type=pl.DeviceIdType.MESH)
        pl.semaphore_signal(barrier, inc=1, device_id=(px, oy),
                            device_id_type=pl.DeviceIdType.MESH)
        pl.semaphore_wait(barrier, 2)

        for c in range(Q // R):
            cp = pltpu.make_async_copy(
                x_hbm.at[pl.ds(py * Q + c * R, R), pl.ds(ox * N_HALF, N_HALF)],
                stage, copy_sem)
            cp.start()
            cp.wait()
            send_buf[pl.ds(c * R, R), :] = stage[...].astype(jnp.bfloat16)

        x_rdma = pltpu.make_async_remote_copy(
            src_ref=send_buf,
            dst_ref=out_ref.at[pl.ds(px * M_PER + py * Q, Q), :],
            send_sem=sx, recv_sem=rx,
            device_id=(ox, py), device_id_type=pl.DeviceIdType.MESH)
        x_rdma.start()

        for c in range(M_PER // R):
            cp = pltpu.make_async_copy(
                x_hbm.at[pl.ds(c * R, R), pl.ds(px * N_HALF, N_HALF)],
                stage, copy_sem)
            cp.start()
            cp.wait()
            out_ref[pl.ds(px * M_PER + c * R, R), :] = (
                stage[...].astype(jnp.bfloat16))

        x_rdma.wait()

        yq = out_ref.at[pl.ds(ox * M_PER + py * Q, Q), :]
        y_rdma = pltpu.make_async_remote_copy(
            src_ref=yq,
            dst_ref=yq,
            send_sem=sy, recv_sem=ry,
            device_id=(px, oy), device_id_type=pl.DeviceIdType.MESH)
        y_rdma.start()
        y_rdma.wait()

    return pl.pallas_call(
        body,
        out_shape=jax.ShapeDtypeStruct((2 * M_PER, N_HALF), jnp.bfloat16),
        in_specs=[pl.BlockSpec(memory_space=pl.ANY)],
        out_specs=pl.BlockSpec(memory_space=pltpu.MemorySpace.VMEM),
        scratch_shapes=[
            pltpu.VMEM((R, N_HALF), jnp.float32),
            pltpu.VMEM((Q, N_HALF), jnp.bfloat16),
            pltpu.SemaphoreType.DMA,
            pltpu.SemaphoreType.DMA,
            pltpu.SemaphoreType.DMA,
            pltpu.SemaphoreType.DMA,
            pltpu.SemaphoreType.DMA,
        ],
        compiler_params=pltpu.CompilerParams(
            collective_id=0,
            vmem_limit_bytes=64 * 1024 * 1024,
        ),
    )(x)


# device time: 145435 ns/iter; 1.5521x vs baseline; 1.5521x over previous
import jax
import jax.numpy as jnp
from jax import lax
from jax.experimental import pallas as pl
from jax.experimental.pallas import tpu as pltpu

M_PER = 8192
N_HALF = 1024
Q = 4096
CH = 256
K = Q // CH
SC = 1024
NS = Q // SC
CPS = SC // CH


def kernel(x):
    def body(x_hbm, out_ref, stage, send_buf, csems, sx, rx, sy, ry):
        px = lax.axis_index("x")
        py = lax.axis_index("y")
        ox = 1 - px
        oy = 1 - py

        barrier = pltpu.get_barrier_semaphore()
        pl.semaphore_signal(barrier, inc=1, device_id=(ox, py),
                            device_id_type=pl.DeviceIdType.MESH)
        pl.semaphore_signal(barrier, inc=1, device_id=(px, oy),
                            device_id_type=pl.DeviceIdType.MESH)
        pl.semaphore_wait(barrier, 2)

        x_rdmas = []

        def convert_and_send(s, cp):
            cp.wait()
            send_buf[pl.ds(s * SC, SC), :] = stage[s % 2].astype(jnp.bfloat16)
            for j in range(CPS):
                k = s * CPS + j
                r = pltpu.make_async_remote_copy(
                    src_ref=send_buf.at[pl.ds(k * CH, CH), :],
                    dst_ref=out_ref.at[pl.ds(px * M_PER + py * Q + k * CH, CH), :],
                    send_sem=sx.at[k], recv_sem=rx.at[k],
                    device_id=(ox, py), device_id_type=pl.DeviceIdType.MESH)
                r.start()
                x_rdmas.append(r)

        cps = []
        for s in range(NS):
            cp = pltpu.make_async_copy(
                x_hbm.at[pl.ds(py * Q + s * SC, SC), pl.ds(ox * N_HALF, N_HALF)],
                stage.at[s % 2], csems.at[s % 2])
            cp.start()
            cps.append(cp)
            if s > 0:
                convert_and_send(s - 1, cps[s - 1])
        convert_and_send(NS - 1, cps[NS - 1])

        local_cps = []

        def convert_local(c):
            local_cps[c].wait()
            out_ref[pl.ds(px * M_PER + c * SC, SC), :] = (
                stage[c % 2].astype(jnp.bfloat16))

        for c in range(M_PER // SC):
            cp = pltpu.make_async_copy(
                x_hbm.at[pl.ds(c * SC, SC), pl.ds(px * N_HALF, N_HALF)],
                stage.at[c % 2], csems.at[c % 2])
            cp.start()
            local_cps.append(cp)
            if c > 0:
                convert_local(c - 1)
        convert_local(M_PER // SC - 1)

        y_rdmas = []
        for k in range(K):
            x_rdmas[k].wait_recv()
            yq = out_ref.at[pl.ds(ox * M_PER + py * Q + k * CH, CH), :]
            yr = pltpu.make_async_remote_copy(
                src_ref=yq, dst_ref=yq,
                send_sem=sy.at[k], recv_sem=ry.at[k],
                device_id=(px, oy), device_id_type=pl.DeviceIdType.MESH)
            yr.start()
            y_rdmas.append(yr)

        for k in range(K):
            x_rdmas[k].wait_send()
            y_rdmas[k].wait()

    return pl.pallas_call(
        body,
        out_shape=jax.ShapeDtypeStruct((2 * M_PER, N_HALF), jnp.bfloat16),
        in_specs=[pl.BlockSpec(memory_space=pl.ANY)],
        out_specs=pl.BlockSpec(memory_space=pltpu.MemorySpace.VMEM),
        scratch_shapes=[
            pltpu.VMEM((2, SC, N_HALF), jnp.float32),
            pltpu.VMEM((Q, N_HALF), jnp.bfloat16),
            pltpu.SemaphoreType.DMA((2,)),
            pltpu.SemaphoreType.DMA((K,)),
            pltpu.SemaphoreType.DMA((K,)),
            pltpu.SemaphoreType.DMA((K,)),
            pltpu.SemaphoreType.DMA((K,)),
        ],
        compiler_params=pltpu.CompilerParams(
            collective_id=0,
            vmem_limit_bytes=64 * 1024 * 1024,
        ),
    )(x)
